# baseline (speedup 1.0000x reference)
import functools

import jax
import jax.numpy as jnp
from jax import lax
from jax.experimental import pallas as pl
from jax.experimental.pallas import tpu as pltpu
from jax.experimental.pallas import tpu_sc as plsc

_COMMIT = 0.25

_TN = 1024
_TK = 1024

_NC = 2
_NS = 16
_CH = 144


def _l2_normalize(v, axis):
    n = jnp.linalg.norm(v, ord=2, axis=axis, keepdims=True)
    return v / jnp.maximum(n, 1e-12)


def _argmin_body(xn_ref, wnt_ref, a_ref, b_ref, m_ref, idx_ref, loss_ref,
                 acc_ref, *, nk, ni, dcount):
    i = pl.program_id(0)
    xn = xn_ref[...]
    xn_bf = xn.astype(jnp.bfloat16)
    a = a_ref[...]
    tn = xn.shape[0]

    def body(t, carry):
        best_d, best_i = carry
        wnt = wnt_ref[:, pl.ds(t * _TK, _TK)]
        s = lax.dot_general(xn_bf, wnt.astype(jnp.bfloat16),
                            (((1,), (0,)), ((), ())),
                            preferred_element_type=jnp.float32)
        s = s.astype(jnp.bfloat16).astype(jnp.float32)
        d = (a + b_ref[:, pl.ds(t * _TK, _TK)]) - 2.0 * s
        lm = jnp.min(d, axis=1, keepdims=True)
        kk = lax.broadcasted_iota(jnp.int32, d.shape, 1) + t * _TK
        li = jnp.min(jnp.where(d == lm, kk, jnp.int32(2**31 - 1)),
                     axis=1, keepdims=True)
        upd = lm < best_d
        return (jnp.where(upd, lm, best_d), jnp.where(upd, li, best_i))

    init = (jnp.full((tn, 1), jnp.inf, jnp.float32),
            jnp.zeros((tn, 1), jnp.int32))
    best_d, best_i = lax.fori_loop(0, nk, body, init)
    idx_ref[...] = best_i

    m = m_ref[...]
    part = jnp.sum(m * best_d)
    cnt = jnp.sum(m)

    @pl.when(i == 0)
    def _():
        acc_ref[0] = part
        acc_ref[1] = cnt

    @pl.when(i > 0)
    def _():
        acc_ref[0] = acc_ref[0] + part
        acc_ref[1] = acc_ref[1] + cnt

    @pl.when(i == ni - 1)
    def _():
        num = acc_ref[0]
        den = jnp.maximum(acc_ref[1] * jnp.float32(dcount), jnp.float32(1.0))
        loss_ref[0] = (1.0 + _COMMIT) * (num / den)


def _tc_argmin(xn2, wnt, a2, b2, m2):
    n, d = xn2.shape
    k = wnt.shape[1]
    ni = n // _TN
    return pl.pallas_call(
        functools.partial(_argmin_body, nk=k // _TK, ni=ni, dcount=d),
        grid=(ni,),
        in_specs=[
            pl.BlockSpec((_TN, d), lambda i: (i, 0)),
            pl.BlockSpec((d, k), lambda i: (0, 0)),
            pl.BlockSpec((_TN, 1), lambda i: (i, 0)),
            pl.BlockSpec((1, k), lambda i: (0, 0)),
            pl.BlockSpec((_TN, 1), lambda i: (i, 0)),
        ],
        out_specs=[
            pl.BlockSpec((_TN, 1), lambda i: (i, 0)),
            pl.BlockSpec(memory_space=pltpu.SMEM),
        ],
        out_shape=[
            jax.ShapeDtypeStruct((n, 1), jnp.int32),
            jax.ShapeDtypeStruct((1,), jnp.float32),
        ],
        scratch_shapes=[pltpu.SMEM((2,), jnp.float32)],
    )(xn2, wnt, a2, b2, m2)


def _sc_gather(wn, idx):
    n = idx.shape[0]
    d = wn.shape[1]
    nw = _NC * _NS
    bw = n // nw
    nch = bw // _CH

    mesh = plsc.VectorSubcoreMesh(core_axis_name="c", subcore_axis_name="s")

    @functools.partial(
        pl.kernel, mesh=mesh,
        out_type=jax.ShapeDtypeStruct((n, d), jnp.float32),
        scratch_types=[
            pltpu.VMEM((bw,), jnp.int32),
            pltpu.VMEM((_CH, d), jnp.float32),
            pltpu.VMEM((_CH, d), jnp.float32),
            pltpu.SemaphoreType.DMA,
            pltpu.SemaphoreType.DMA,
        ],
    )
    def gather_k(wn_hbm, idx_hbm, out_hbm, idx_v, rows0, rows1, sem0, sem1):
        wid = lax.axis_index("s") * _NC + lax.axis_index("c")
        base = wid * bw
        pltpu.sync_copy(idx_hbm.at[pl.ds(base, bw)], idx_v)
        rows = (rows0, rows1)
        sems = (sem0, sem1)
        cps = [None, None]
        cps[0] = pltpu.async_copy(
            wn_hbm.at[idx_v.at[pl.ds(0, _CH)]], rows[0], sems[0])
        for c in range(nch):
            cur = c % 2
            nxt = (c + 1) % 2
            if c + 1 < nch:
                cps[nxt] = pltpu.async_copy(
                    wn_hbm.at[idx_v.at[pl.ds((c + 1) * _CH, _CH)]],
                    rows[nxt], sems[nxt])
            cps[cur].wait()
            pltpu.sync_copy(rows[cur], out_hbm.at[pl.ds(base + c * _CH, _CH)])

    return gather_k(wn, idx)


def kernel(x, mask, W):
    bb, ll, d = x.shape
    k = W.shape[0]
    n = bb * ll

    xn = _l2_normalize(x, -1)
    wn = _l2_normalize(W, -1)
    a = jnp.sum(xn ** 2, axis=-1, keepdims=True)
    b = jnp.sum(wn ** 2, axis=-1)

    xn2 = xn.reshape(n, d)
    a2 = a.reshape(n, 1)
    b2 = b.reshape(1, k)
    m2 = mask.reshape(n, 1).astype(jnp.float32)
    wnt = wn.T

    idx, loss1 = _tc_argmin(xn2, wnt, a2, b2, m2)
    quant = jnp.take(wn, idx.reshape(n), axis=0)
    return (quant.reshape(bb, ll, d), loss1[0])


# speedup vs baseline: 1.0999x; 1.0999x over previous
import functools

import jax
import jax.numpy as jnp
from jax import lax
from jax.experimental import pallas as pl
from jax.experimental.pallas import tpu as pltpu
from jax.experimental.pallas import tpu_sc as plsc

_COMMIT = 0.25

_TN = 2048

_NC = 2
_NS = 16
_CH = 144


def _l2_normalize(v, axis):
    n = jnp.linalg.norm(v, ord=2, axis=axis, keepdims=True)
    return v / jnp.maximum(n, 1e-12)


def _row_norm(v):
    n = jnp.sqrt(jnp.sum(v * v, axis=1, keepdims=True))
    return v / jnp.maximum(n, 1e-12)


def _norm_loss_body(x_ref, g_ref, m_ref, q_ref, loss_ref, acc_ref, *, ni, dcount):
    i = pl.program_id(0)
    xn = _row_norm(x_ref[...])
    qn = _row_norm(g_ref[...])
    q_ref[...] = qn
    diff = qn - xn
    m = m_ref[...]
    part = jnp.sum((diff * diff) * m)
    cnt = jnp.sum(m)

    @pl.when(i == 0)
    def _():
        acc_ref[0] = part
        acc_ref[1] = cnt

    @pl.when(i > 0)
    def _():
        acc_ref[0] = acc_ref[0] + part
        acc_ref[1] = acc_ref[1] + cnt

    @pl.when(i == ni - 1)
    def _():
        num = acc_ref[0]
        den = jnp.maximum(acc_ref[1] * jnp.float32(dcount), jnp.float32(1.0))
        loss_ref[0] = (1.0 + _COMMIT) * (num / den)


def _tc_norm_loss(x2, g2, m2):
    n, d = x2.shape
    ni = n // _TN
    return pl.pallas_call(
        functools.partial(_norm_loss_body, ni=ni, dcount=d),
        grid=(ni,),
        in_specs=[
            pl.BlockSpec((_TN, d), lambda i: (i, 0)),
            pl.BlockSpec((_TN, d), lambda i: (i, 0)),
            pl.BlockSpec((_TN, 1), lambda i: (i, 0)),
        ],
        out_specs=[
            pl.BlockSpec((_TN, d), lambda i: (i, 0)),
            pl.BlockSpec(memory_space=pltpu.SMEM),
        ],
        out_shape=[
            jax.ShapeDtypeStruct((n, d), jnp.float32),
            jax.ShapeDtypeStruct((1,), jnp.float32),
        ],
        scratch_shapes=[pltpu.SMEM((2,), jnp.float32)],
    )(x2, g2, m2)


def _sc_gather(table, idx):
    n = idx.shape[0]
    d = table.shape[1]
    nw = _NC * _NS
    bw = n // nw
    nch = bw // _CH

    mesh = plsc.VectorSubcoreMesh(core_axis_name="c", subcore_axis_name="s")

    @functools.partial(
        pl.kernel, mesh=mesh,
        out_type=jax.ShapeDtypeStruct((n, d), jnp.float32),
        scratch_types=[
            pltpu.VMEM((bw,), jnp.int32),
            pltpu.VMEM((_CH, d), jnp.float32),
            pltpu.VMEM((_CH, d), jnp.float32),
            pltpu.SemaphoreType.DMA,
            pltpu.SemaphoreType.DMA,
        ],
    )
    def gather_k(tab_hbm, idx_hbm, out_hbm, idx_v, rows0, rows1, sem0, sem1):
        wid = lax.axis_index("s") * _NC + lax.axis_index("c")
        base = wid * bw
        pltpu.sync_copy(idx_hbm.at[pl.ds(base, bw)], idx_v)
        rows = (rows0, rows1)
        sems = (sem0, sem1)
        cps = [None, None]
        cps[0] = pltpu.async_copy(
            tab_hbm.at[idx_v.at[pl.ds(0, _CH)]], rows[0], sems[0])
        for c in range(nch):
            cur = c % 2
            nxt = (c + 1) % 2
            if c + 1 < nch:
                cps[nxt] = pltpu.async_copy(
                    tab_hbm.at[idx_v.at[pl.ds((c + 1) * _CH, _CH)]],
                    rows[nxt], sems[nxt])
            cps[cur].wait()
            pltpu.sync_copy(rows[cur], out_hbm.at[pl.ds(base + c * _CH, _CH)])

    return gather_k(table, idx)


def kernel(x, mask, W):
    bb, ll, d = x.shape
    n = bb * ll

    xn = _l2_normalize(x, -1)
    wn = _l2_normalize(W, -1)
    distances = (
        jnp.sum(xn ** 2, axis=-1, keepdims=True)
        + jnp.sum(wn ** 2, axis=-1)[None, None, :]
        - 2.0 * jnp.einsum('nld,kd->nlk', xn, wn)
    )
    idx = jnp.argmin(distances, axis=-1).reshape(n)

    graw = _sc_gather(W, idx)
    m2 = mask.reshape(n, 1).astype(jnp.float32)
    q2, loss1 = _tc_norm_loss(x.reshape(n, d), graw, m2)
    quantized = q2.reshape(bb, ll, d)
    quantized_out = xn + lax.stop_gradient(quantized - xn)
    return (quantized_out, loss1[0])
